# baseline (speedup 1.0000x reference)
import jax
import jax.numpy as jnp
from jax.experimental import pallas as pl
from jax.experimental.pallas import tpu as pltpu
from jax.experimental.pallas import tpu_sc as plsc

B = 4096
F = 26
V = 100000
E = 16
PAIRS = F * (F - 1) // 2
DNN_IN = PAIRS * E
BF = B * F
ROWS2 = F * V // 8
ROWS1 = (F * V + 64) // 128
GW = 128
BLK = 256


def _sc_gather(emb2v, emb1v, idx2, idx1):
    mesh = plsc.VectorSubcoreMesh(core_axis_name="core", subcore_axis_name="subcore")

    @pl.kernel(
        out_type=(
            jax.ShapeDtypeStruct((BF, 128), jnp.float32),
            jax.ShapeDtypeStruct((BF, 128), jnp.float32),
        ),
        mesh=mesh,
    )
    def k(e2_hbm, e1_hbm, i2_hbm, i1_hbm, o2_hbm, o1_hbm):
        def body(i2_vmem, i1_vmem, o2_vmem, o1_vmem):
            pltpu.sync_copy(e2_hbm.at[i2_vmem.at[0]], o2_vmem)
            pltpu.sync_copy(e1_hbm.at[i1_vmem.at[0]], o1_vmem)

        pltpu.emit_pipeline(
            body,
            grid=(BF // GW,),
            in_specs=[
                pl.BlockSpec((1, GW), lambda i: (0, i)),
                pl.BlockSpec((1, GW), lambda i: (0, i)),
            ],
            out_specs=[
                pl.BlockSpec((GW, 128), lambda i: (i, 0)),
                pl.BlockSpec((GW, 128), lambda i: (i, 0)),
            ],
            core_axis_name=("core", "subcore"),
            dimension_semantics=(pltpu.PARALLEL,),
        )(i2_hbm, i1_hbm, o2_hbm, o1_hbm)

    return k(emb2v, emb1v, idx2, idx1)


def _tc_body(e2_ref, l1_ref, chunk_ref, lane_ref, dense_ref, Wld_ref, bld_ref,
             W1_ref, b1_ref, W2_ref, b2_ref, W3_ref, b3_ref, W4_ref, b4_ref,
             Wf_ref, bf_ref, Wl_ref, bl_ref, fin_ref, like_ref):
    xg = e2_ref[...]
    chunk = chunk_ref[...]
    feats = []
    for f in range(F):
        row = xg[:, f * 128:(f + 1) * 128]
        c = chunk[:, f:f + 1]
        sel = jnp.zeros((row.shape[0], E), jnp.float32)
        for k in range(8):
            sel = sel + jnp.where(c == k, row[:, k * E:(k + 1) * E], 0.0)
        feats.append(sel)
    x = jnp.concatenate(feats, axis=1)

    pieces = []
    for i in range(F - 1):
        xi = x[:, i * E:(i + 1) * E]
        rest = x[:, (i + 1) * E:]
        rep = jnp.concatenate([xi] * (F - 1 - i), axis=1)
        pieces.append(rep * rest)
    prods = jnp.concatenate(pieces, axis=1)
    h = jnp.dot(prods, W1_ref[...], preferred_element_type=jnp.float32)
    h = jnp.maximum(h + b1_ref[...], 0.0)
    h = jnp.dot(h, W2_ref[...], preferred_element_type=jnp.float32)
    h = jnp.maximum(h + b2_ref[...], 0.0)
    h = jnp.dot(h, W3_ref[...], preferred_element_type=jnp.float32)
    h = jnp.maximum(h + b3_ref[...], 0.0)
    dnn = jnp.dot(h, W4_ref[...], preferred_element_type=jnp.float32) + b4_ref[...]

    l1 = l1_ref[...]
    lane = lane_ref[...]
    iota = jax.lax.broadcasted_iota(jnp.int32, (1, 128), 1)
    linsum = jnp.zeros((l1.shape[0], 1), jnp.float32)
    for f in range(F):
        row = l1[:, f * 128:(f + 1) * 128]
        sel = jnp.where(lane[:, f:f + 1] == iota, row, 0.0)
        linsum = linsum + jnp.sum(sel, axis=1, keepdims=True)
    first = jnp.dot(dense_ref[...], Wld_ref[...],
                    preferred_element_type=jnp.float32) + bld_ref[...] + linsum

    logits = first + dnn
    fin_ref[...] = jax.nn.sigmoid(logits * Wf_ref[0, 0] + bf_ref[0, 0])
    like_ref[...] = jax.nn.sigmoid(logits * Wl_ref[0, 0] + bl_ref[0, 0])


def _tc_specs():
    def blk(shape):
        return pl.BlockSpec(shape, lambda i: (i, 0))

    def whole(shape):
        return pl.BlockSpec(shape, lambda i: (0, 0))

    in_specs = [
        blk((BLK, F * 128)),
        blk((BLK, F * 128)),
        blk((BLK, F)),
        blk((BLK, F)),
        blk((BLK, 13)),
        whole((13, 1)), whole((1, 1)),
        whole((DNN_IN, 200)), whole((1, 200)),
        whole((200, 200)), whole((1, 200)),
        whole((200, 200)), whole((1, 200)),
        whole((200, 1)), whole((1, 1)),
        whole((1, 1)), whole((1, 1)),
        whole((1, 1)), whole((1, 1)),
    ]
    out_specs = [blk((BLK, 1)), blk((BLK, 1))]
    return in_specs, out_specs


def _tc_forward(e2g, l1g, chunk, lane, dense, Wld, bld, W1, b1, W2, b2, W3, b3,
                W4, b4, Wf, bf, Wl, bl):
    in_specs, out_specs = _tc_specs()
    return pl.pallas_call(
        _tc_body,
        grid=(B // BLK,),
        in_specs=in_specs,
        out_specs=out_specs,
        out_shape=(
            jax.ShapeDtypeStruct((B, 1), jnp.float32),
            jax.ShapeDtypeStruct((B, 1), jnp.float32),
        ),
    )(e2g, l1g, chunk, lane, dense, Wld, bld, W1, b1, W2, b2, W3, b3, W4, b4,
      Wf, bf, Wl, bl)


def kernel(sparse_inputs, dense_inputs, emb1, emb2, W_ld, b_ld,
           W1, b1, W2, b2, W3, b3, W4, b4, Wf, bf, Wl, bl):
    si = sparse_inputs.astype(jnp.int32)
    offs = (jnp.arange(F, dtype=jnp.int32) * V)[None, :]
    flat = si + offs
    row2 = flat // 8
    chunk = flat % 8
    row1 = flat // 128
    lane = flat % 128

    emb2v = emb2.reshape(ROWS2, 128)
    emb1v = jnp.concatenate(
        [emb1.reshape(1, F * V), jnp.zeros((1, 64), jnp.float32)], axis=1
    ).reshape(ROWS1, 128)

    e2rows, l1rows = _sc_gather(emb2v, emb1v,
                                row2.reshape(1, BF), row1.reshape(1, BF))
    e2g = e2rows.reshape(B, F * 128)
    l1g = l1rows.reshape(B, F * 128)

    return _tc_forward(
        e2g, l1g, chunk, lane, dense_inputs, W_ld, b_ld.reshape(1, 1),
        W1, b1.reshape(1, 200), W2, b2.reshape(1, 200), W3, b3.reshape(1, 200),
        W4, b4.reshape(1, 1), Wf, bf.reshape(1, 1), Wl, bl.reshape(1, 1))


# speedup vs baseline: 1.4131x; 1.4131x over previous
import jax
import jax.numpy as jnp
from jax.experimental import pallas as pl
from jax.experimental.pallas import tpu as pltpu
from jax.experimental.pallas import tpu_sc as plsc

B = 4096
F = 26
V = 100000
E = 16
PAIRS = F * (F - 1) // 2
DNN_IN = PAIRS * E
BF = B * F
ROWS2 = F * V
ROWS1 = F * V // 16
GW = 128
BLK = 256


def _sc_gather(emb2v, emb1v, idx2, idx1):
    mesh = plsc.VectorSubcoreMesh(core_axis_name="core", subcore_axis_name="subcore")

    @pl.kernel(
        out_type=(
            jax.ShapeDtypeStruct((BF, E), jnp.float32),
            jax.ShapeDtypeStruct((BF, 16), jnp.float32),
        ),
        mesh=mesh,
        compiler_params=pltpu.CompilerParams(use_tc_tiling_on_sc=False),
    )
    def k(e2_hbm, e1_hbm, i2_hbm, i1_hbm, o2_hbm, o1_hbm):
        def body(i2_vmem, i1_vmem, o2_vmem, o1_vmem):
            pltpu.sync_copy(e2_hbm.at[i2_vmem.at[0]], o2_vmem)
            pltpu.sync_copy(e1_hbm.at[i1_vmem.at[0]], o1_vmem)

        pltpu.emit_pipeline(
            body,
            grid=(BF // GW,),
            in_specs=[
                pl.BlockSpec((1, GW), lambda i: (0, i)),
                pl.BlockSpec((1, GW), lambda i: (0, i)),
            ],
            out_specs=[
                pl.BlockSpec((GW, E), lambda i: (i, 0)),
                pl.BlockSpec((GW, 16), lambda i: (i, 0)),
            ],
            core_axis_name=("core", "subcore"),
            dimension_semantics=(pltpu.PARALLEL,),
        )(i2_hbm, i1_hbm, o2_hbm, o1_hbm)

    return k(emb2v, emb1v, idx2, idx1)


def _tc_body(e2_ref, l1_ref, lane_ref, dense_ref, Wld_ref, bld_ref,
             W1_ref, b1_ref, W2_ref, b2_ref, W3_ref, b3_ref, W4_ref, b4_ref,
             Wf_ref, bf_ref, Wl_ref, bl_ref, fin_ref, like_ref):
    x = e2_ref[...]

    pieces = []
    for i in range(F - 1):
        xi = x[:, i * E:(i + 1) * E]
        rest = x[:, (i + 1) * E:]
        rep = jnp.concatenate([xi] * (F - 1 - i), axis=1)
        pieces.append(rep * rest)
    prods = jnp.concatenate(pieces, axis=1)
    h = jnp.dot(prods, W1_ref[...], preferred_element_type=jnp.float32)
    h = jnp.maximum(h + b1_ref[...], 0.0)
    h = jnp.dot(h, W2_ref[...], preferred_element_type=jnp.float32)
    h = jnp.maximum(h + b2_ref[...], 0.0)
    h = jnp.dot(h, W3_ref[...], preferred_element_type=jnp.float32)
    h = jnp.maximum(h + b3_ref[...], 0.0)
    dnn = jnp.dot(h, W4_ref[...], preferred_element_type=jnp.float32) + b4_ref[...]

    l1 = l1_ref[...]
    lane = lane_ref[...]
    iota = jax.lax.broadcasted_iota(jnp.int32, (1, 16), 1)
    linsum = jnp.zeros((l1.shape[0], 1), jnp.float32)
    for f in range(F):
        row = l1[:, f * 16:(f + 1) * 16]
        sel = jnp.where(lane[:, f:f + 1] == iota, row, 0.0)
        linsum = linsum + jnp.sum(sel, axis=1, keepdims=True)
    first = jnp.dot(dense_ref[...], Wld_ref[...],
                    preferred_element_type=jnp.float32) + bld_ref[...] + linsum

    logits = first + dnn
    fin_ref[...] = jax.nn.sigmoid(logits * Wf_ref[0, 0] + bf_ref[0, 0])
    like_ref[...] = jax.nn.sigmoid(logits * Wl_ref[0, 0] + bl_ref[0, 0])


def _tc_specs():
    def blk(shape):
        return pl.BlockSpec(shape, lambda i: (i, 0))

    def whole(shape):
        return pl.BlockSpec(shape, lambda i: (0, 0))

    in_specs = [
        blk((BLK, F * E)),
        blk((BLK, F * 16)),
        blk((BLK, F)),
        blk((BLK, 13)),
        whole((13, 1)), whole((1, 1)),
        whole((DNN_IN, 200)), whole((1, 200)),
        whole((200, 200)), whole((1, 200)),
        whole((200, 200)), whole((1, 200)),
        whole((200, 1)), whole((1, 1)),
        whole((1, 1)), whole((1, 1)),
        whole((1, 1)), whole((1, 1)),
    ]
    out_specs = [blk((BLK, 1)), blk((BLK, 1))]
    return in_specs, out_specs


def _tc_forward(e2g, l1g, lane, dense, Wld, bld, W1, b1, W2, b2, W3, b3,
                W4, b4, Wf, bf, Wl, bl):
    in_specs, out_specs = _tc_specs()
    return pl.pallas_call(
        _tc_body,
        grid=(B // BLK,),
        in_specs=in_specs,
        out_specs=out_specs,
        out_shape=(
            jax.ShapeDtypeStruct((B, 1), jnp.float32),
            jax.ShapeDtypeStruct((B, 1), jnp.float32),
        ),
    )(e2g, l1g, lane, dense, Wld, bld, W1, b1, W2, b2, W3, b3, W4, b4,
      Wf, bf, Wl, bl)


def kernel(sparse_inputs, dense_inputs, emb1, emb2, W_ld, b_ld,
           W1, b1, W2, b2, W3, b3, W4, b4, Wf, bf, Wl, bl):
    si = sparse_inputs.astype(jnp.int32)
    offs = (jnp.arange(F, dtype=jnp.int32) * V)[None, :]
    flat = si + offs
    row1 = flat // 16
    lane = flat % 16

    emb2v = emb2.reshape(ROWS2, E)
    emb1v = emb1.reshape(ROWS1, 16)

    e2rows, l1rows = _sc_gather(emb2v, emb1v,
                                flat.reshape(1, BF), row1.reshape(1, BF))
    e2g = e2rows.reshape(B, F * E)
    l1g = l1rows.reshape(B, F * 16)

    return _tc_forward(
        e2g, l1g, lane, dense_inputs, W_ld, b_ld.reshape(1, 1),
        W1, b1.reshape(1, 200), W2, b2.reshape(1, 200), W3, b3.reshape(1, 200),
        W4, b4.reshape(1, 1), Wf, bf.reshape(1, 1), Wl, bl.reshape(1, 1))
